# Initial kernel scaffold; baseline (speedup 1.0000x reference)
import jax
import jax.numpy as jnp
from jax.experimental import pallas as pl


def kernel(x, edge_index, W1, b1, W2, b2, W3, b3):
    raise NotImplementedError("write your pallas kernel here")


# speedup vs baseline: 2.5849x; 2.5849x over previous
import functools

import jax
import jax.numpy as jnp
from jax import lax
from jax.experimental import pallas as pl
from jax.experimental.pallas import tpu as pltpu
from jax.experimental.pallas import tpu_sc as plsc

N = 10000
E = 320000
D = 128
NC = 2
NS = 16
NW = NC * NS
CHUNK = 128
NPAD = 10240
RPB = NPAD // NW
ACC_ROWS = RPB + 8
DUMMY = RPB
CAP = 12800

_MESH = plsc.VectorSubcoreMesh(
    core_axis_name="c", subcore_axis_name="s", num_cores=NC, num_subcores=NS)


@functools.partial(
    pl.kernel,
    out_type=jax.ShapeDtypeStruct((NPAD, 16), jnp.float32),
    mesh=_MESH,
    scratch_types=[
        pltpu.VMEM((CHUNK,), jnp.int32),
        pltpu.VMEM((ACC_ROWS, 16), jnp.float32),
        pltpu.VMEM((NW + 16,), jnp.int32),
    ],
)
def _deg_kernel(dstl_hbm, cnt_hbm, out_hbm, dst_v, acc, cnt_v):
    c = lax.axis_index("c")
    s = lax.axis_index("s")
    wid = s * NC + c

    zero16 = jnp.zeros((16,), jnp.float32)

    def z(i, _):
        acc[i] = zero16
        return 0
    lax.fori_loop(0, ACC_ROWS, z, 0)

    one0 = jnp.where(lax.broadcasted_iota(jnp.int32, (16,), 0) == 0, 1.0, 0.0)
    pltpu.sync_copy(cnt_hbm, cnt_v)
    n_t = cnt_v[pl.ds(wid, 16)][0]
    nchunks = (n_t + CHUNK - 1) // CHUNK
    base = wid * CAP

    def chunk_body(j, _):
        pltpu.sync_copy(dstl_hbm.at[pl.ds(base + j * CHUNK, CHUNK)], dst_v)

        def grp_body(g, _2):
            dvec = dst_v[pl.ds(g * 16, 16)]
            for i in range(16):
                r = dvec[i]
                acc[r] = acc[r] + one0
            return 0
        lax.fori_loop(0, CHUNK // 16, grp_body, 0)
        return 0
    lax.fori_loop(0, nchunks, chunk_body, 0)
    pltpu.sync_copy(acc.at[pl.ds(0, RPB)], out_hbm.at[pl.ds(wid * RPB, RPB)])


@functools.partial(
    pl.kernel,
    out_type=jax.ShapeDtypeStruct((NPAD, D), jnp.float32),
    mesh=_MESH,
    scratch_types=[
        pltpu.VMEM((CHUNK,), jnp.int32),
        pltpu.VMEM((CHUNK,), jnp.int32),
        pltpu.VMEM((CHUNK, D), jnp.float32),
        pltpu.VMEM((ACC_ROWS, D), jnp.float32),
        pltpu.VMEM((NW + 16,), jnp.int32),
        pltpu.SemaphoreType.DMA,
    ],
)
def _spmm_kernel(g_hbm, src_hbm, dstl_hbm, cnt_hbm, out_hbm,
                 src_v, dst_v, rows_v, acc, cnt_v, sem):
    c = lax.axis_index("c")
    s = lax.axis_index("s")
    wid = s * NC + c

    zero16 = jnp.zeros((16,), jnp.float32)

    def z(i, _):
        for k in range(D // 16):
            acc[i, pl.ds(k * 16, 16)] = zero16
        return 0
    lax.fori_loop(0, ACC_ROWS, z, 0)

    pltpu.sync_copy(cnt_hbm, cnt_v)
    n_t = cnt_v[pl.ds(wid, 16)][0]
    nchunks = (n_t + CHUNK - 1) // CHUNK
    base = wid * CAP

    def chunk_body(j, _):
        off = base + j * CHUNK
        pltpu.sync_copy(src_hbm.at[pl.ds(off, CHUNK)], src_v)
        pltpu.sync_copy(dstl_hbm.at[pl.ds(off, CHUNK)], dst_v)
        pltpu.async_copy(g_hbm.at[src_v], rows_v, sem).wait()

        def grp_body(g, _2):
            dvec = dst_v[pl.ds(g * 16, 16)]
            for i in range(16):
                r = dvec[i]
                e = g * 16 + i
                for k in range(D // 16):
                    sl = pl.ds(k * 16, 16)
                    acc[r, sl] = acc[r, sl] + rows_v[e, sl]
            return 0
        lax.fori_loop(0, CHUNK // 16, grp_body, 0)
        return 0
    lax.fori_loop(0, nchunks, chunk_body, 0)
    pltpu.sync_copy(acc.at[pl.ds(0, RPB)], out_hbm.at[pl.ds(wid * RPB, RPB)])


_BS = 2560
_GRID = NPAD // _BS


def _dinv_of(degp):
    return lax.rsqrt(degp[:, 0] + 1.0)


def _tc_first_body(x_ref, w_ref, degp_ref, g_ref):
    h = jnp.dot(x_ref[...], w_ref[...], preferred_element_type=jnp.float32)
    dinv = _dinv_of(degp_ref[...])
    g_ref[...] = h * dinv[:, None]


def _tc_mid_body(degp_ref, s_ref, g_ref, w_ref, b_ref, gn_ref):
    dinv = _dinv_of(degp_ref[...])
    ssum = s_ref[...] + g_ref[...]
    o = jnp.maximum(ssum * dinv[:, None] + b_ref[...], 0.0)
    h = jnp.dot(o, w_ref[...], preferred_element_type=jnp.float32)
    gn_ref[...] = h * dinv[:, None]


def _tc_last_body(degp_ref, s_ref, g_ref, b_ref, o_ref):
    dinv = _dinv_of(degp_ref[...])
    ssum = s_ref[...] + g_ref[...]
    o_ref[...] = ssum * dinv[:, None] + b_ref[...]


def _rows_spec(width):
    return pl.BlockSpec((_BS, width), lambda i: (i, 0))


def _full_spec(r, cols):
    return pl.BlockSpec((r, cols), lambda i: (0, 0))


_tc_first = pl.pallas_call(
    _tc_first_body,
    grid=(_GRID,),
    in_specs=[_rows_spec(D), _full_spec(D, D), _rows_spec(16)],
    out_specs=_rows_spec(D),
    out_shape=jax.ShapeDtypeStruct((NPAD, D), jnp.float32),
)

_tc_mid = pl.pallas_call(
    _tc_mid_body,
    grid=(_GRID,),
    in_specs=[_rows_spec(16), _rows_spec(D), _rows_spec(D),
              _full_spec(D, D), _full_spec(1, D)],
    out_specs=_rows_spec(D),
    out_shape=jax.ShapeDtypeStruct((NPAD, D), jnp.float32),
)

_tc_last = pl.pallas_call(
    _tc_last_body,
    grid=(_GRID,),
    in_specs=[_rows_spec(16), _rows_spec(D), _rows_spec(D), _full_spec(1, D)],
    out_specs=_rows_spec(D),
    out_shape=jax.ShapeDtypeStruct((NPAD, D), jnp.float32),
)


def kernel(x, edge_index, W1, b1, W2, b2, W3, b3):
    src = edge_index[0].astype(jnp.int32)
    dst = edge_index[1].astype(jnp.int32)

    order = jnp.argsort(dst)
    ds_ = jnp.take(dst, order)
    sr_ = jnp.take(src, order)
    binid = ds_ // RPB
    starts = jnp.searchsorted(ds_, (jnp.arange(NW) * RPB).astype(ds_.dtype))
    starts = starts.astype(jnp.int32)
    counts = jnp.diff(jnp.append(starts, jnp.int32(E)))
    pos = jnp.arange(E, dtype=jnp.int32) - jnp.take(starts, binid)
    slot = binid * CAP + pos
    src_pad = jnp.zeros((NW * CAP,), jnp.int32).at[slot].set(sr_, mode="drop")
    dstl_pad = jnp.full((NW * CAP,), DUMMY, jnp.int32).at[slot].set(
        ds_ - binid * RPB, mode="drop")
    cnts = jnp.concatenate([jnp.minimum(counts, CAP).astype(jnp.int32),
                            jnp.zeros((16,), jnp.int32)])

    xp = jnp.pad(x, ((0, NPAD - N), (0, 0)))
    b1r, b2r, b3r = (b.reshape(1, D) for b in (b1, b2, b3))

    degp = _deg_kernel(dstl_pad, cnts)
    g1 = _tc_first(xp, W1, degp)
    s1 = _spmm_kernel(g1, src_pad, dstl_pad, cnts)
    g2 = _tc_mid(degp, s1, g1, W2, b1r)
    s2 = _spmm_kernel(g2, src_pad, dstl_pad, cnts)
    g3 = _tc_mid(degp, s2, g2, W3, b2r)
    s3 = _spmm_kernel(g3, src_pad, dstl_pad, cnts)
    out = _tc_last(degp, s3, g3, b3r)
    return out[:N]


# speedup vs baseline: 6.7547x; 2.6132x over previous
import functools

import jax
import jax.numpy as jnp
from jax import lax
from jax.experimental import pallas as pl
from jax.experimental.pallas import tpu as pltpu
from jax.experimental.pallas import tpu_sc as plsc

N = 10000
E = 320000
D = 128
NC = 2
NS = 16
NW = NC * NS
CHUNK = 128
BLK = 2048
CPB = BLK // CHUNK
NPAD = 10240
RPB = NPAD // NW
ACC_ROWS = RPB + 8
DUMMY = RPB
EPAD = E + BLK + CHUNK

_MESH = plsc.VectorSubcoreMesh(
    core_axis_name="c", subcore_axis_name="s", num_cores=NC, num_subcores=NS)


@functools.partial(
    pl.kernel,
    out_type=jax.ShapeDtypeStruct((NPAD, 16), jnp.float32),
    mesh=_MESH,
    scratch_types=[
        pltpu.VMEM((BLK,), jnp.int32),
        pltpu.VMEM((ACC_ROWS, 16), jnp.float32),
        pltpu.VMEM((NW + 17,), jnp.int32),
    ],
)
def _deg_kernel(dst_hbm, bnd_hbm, out_hbm, dst_b, acc, bnd_v):
    c = lax.axis_index("c")
    s = lax.axis_index("s")
    wid = s * NC + c
    lo = wid * RPB
    hi = lo + RPB

    zero16 = jnp.zeros((16,), jnp.float32)

    def z(i, _):
        acc[i] = zero16
        return 0
    lax.fori_loop(0, ACC_ROWS, z, 0)

    one0 = jnp.where(lax.broadcasted_iota(jnp.int32, (16,), 0) == 0, 1.0, 0.0)
    pltpu.sync_copy(bnd_hbm, bnd_v)
    start = bnd_v[pl.ds(wid, 16)][0]
    end = bnd_v[pl.ds(wid + 1, 16)][0]
    s8 = (start // 8) * 8
    nblocks = (end - s8 + BLK - 1) // BLK

    def block_body(b, _):
        boff = s8 + b * BLK
        pltpu.sync_copy(dst_hbm.at[pl.ds(boff, BLK)], dst_b)

        def grp_body(g, _2):
            dvec = dst_b[pl.ds(g * 16, 16)]
            inr = (dvec >= lo) & (dvec < hi)
            dl = jnp.where(inr, dvec - lo, DUMMY)
            for i in range(16):
                r = dl[i]
                acc[r] = acc[r] + one0
            return 0
        lax.fori_loop(0, BLK // 16, grp_body, 0)
        return 0
    lax.fori_loop(0, nblocks, block_body, 0)
    pltpu.sync_copy(acc.at[pl.ds(0, RPB)], out_hbm.at[pl.ds(wid * RPB, RPB)])


@functools.partial(
    pl.kernel,
    out_type=jax.ShapeDtypeStruct((NPAD, D), jnp.float32),
    mesh=_MESH,
    scratch_types=[
        pltpu.VMEM((BLK,), jnp.int32),
        pltpu.VMEM((BLK,), jnp.int32),
        pltpu.VMEM((CHUNK, D), jnp.float32),
        pltpu.VMEM((CHUNK, D), jnp.float32),
        pltpu.VMEM((ACC_ROWS, D), jnp.float32),
        pltpu.VMEM((NW + 17,), jnp.int32),
        pltpu.SemaphoreType.DMA,
        pltpu.SemaphoreType.DMA,
    ],
)
def _spmm_kernel(g_hbm, src_hbm, dst_hbm, bnd_hbm, out_hbm,
                 src_b, dst_b, rows0, rows1, acc, bnd_v, sem0, sem1):
    c = lax.axis_index("c")
    s = lax.axis_index("s")
    wid = s * NC + c
    lo = wid * RPB
    hi = lo + RPB

    zero16 = jnp.zeros((16,), jnp.float32)

    def z(i, _):
        for k in range(D // 16):
            acc[i, pl.ds(k * 16, 16)] = zero16
        return 0
    lax.fori_loop(0, ACC_ROWS, z, 0)

    pltpu.sync_copy(bnd_hbm, bnd_v)
    start = bnd_v[pl.ds(wid, 16)][0]
    end = bnd_v[pl.ds(wid + 1, 16)][0]
    s8 = (start // 8) * 8
    nchunks = (end - s8 + CHUNK - 1) // CHUNK
    nblocks = (nchunks + CPB - 1) // CPB

    def accumulate(rows_v, cbase):
        def grp_body(g, _2):
            dvec = dst_b[pl.ds(cbase + g * 16, 16)]
            inr = (dvec >= lo) & (dvec < hi)
            dl = jnp.where(inr, dvec - lo, DUMMY)
            for i in range(16):
                r = dl[i]
                e = g * 16 + i
                for k in range(D // 16):
                    sl = pl.ds(k * 16, 16)
                    acc[r, sl] = acc[r, sl] + rows_v[e, sl]
            return 0
        lax.fori_loop(0, CHUNK // 16, grp_body, 0)

    def gather(cidx, rows_v, sem):
        return pltpu.async_copy(
            g_hbm.at[src_b.at[pl.ds(cidx * CHUNK, CHUNK)]], rows_v, sem)

    def block_body(b, _):
        boff = s8 + b * BLK
        pltpu.sync_copy(src_hbm.at[pl.ds(boff, BLK)], src_b)
        pltpu.sync_copy(dst_hbm.at[pl.ds(boff, BLK)], dst_b)
        rem = nchunks - b * CPB
        nch = jnp.minimum(rem, CPB)
        pairs = (nch + 1) // 2
        gather(0, rows0, sem0)

        def pair_body(p, _2):
            gather(2 * p + 1, rows1, sem1)
            pltpu.make_async_copy(
                g_hbm.at[src_b.at[pl.ds(0, CHUNK)]], rows0, sem0).wait()
            accumulate(rows0, 2 * p * CHUNK)

            @pl.when(p + 1 < pairs)
            def _():
                gather(2 * p + 2, rows0, sem0)
            pltpu.make_async_copy(
                g_hbm.at[src_b.at[pl.ds(0, CHUNK)]], rows1, sem1).wait()
            accumulate(rows1, (2 * p + 1) * CHUNK)
            return 0
        lax.fori_loop(0, pairs, pair_body, 0)
        return 0
    lax.fori_loop(0, nblocks, block_body, 0)
    pltpu.sync_copy(acc.at[pl.ds(0, RPB)], out_hbm.at[pl.ds(wid * RPB, RPB)])


_BS = 2560
_GRID = NPAD // _BS


def _dinv_of(degp):
    return lax.rsqrt(degp[:, 0] + 1.0)


def _tc_first_body(x_ref, w_ref, degp_ref, g_ref):
    h = jnp.dot(x_ref[...], w_ref[...], preferred_element_type=jnp.float32)
    dinv = _dinv_of(degp_ref[...])
    g_ref[...] = h * dinv[:, None]


def _tc_mid_body(degp_ref, s_ref, g_ref, w_ref, b_ref, gn_ref):
    dinv = _dinv_of(degp_ref[...])
    ssum = s_ref[...] + g_ref[...]
    o = jnp.maximum(ssum * dinv[:, None] + b_ref[...], 0.0)
    h = jnp.dot(o, w_ref[...], preferred_element_type=jnp.float32)
    gn_ref[...] = h * dinv[:, None]


def _tc_last_body(degp_ref, s_ref, g_ref, b_ref, o_ref):
    dinv = _dinv_of(degp_ref[...])
    ssum = s_ref[...] + g_ref[...]
    o_ref[...] = ssum * dinv[:, None] + b_ref[...]


def _rows_spec(width):
    return pl.BlockSpec((_BS, width), lambda i: (i, 0))


def _full_spec(r, cols):
    return pl.BlockSpec((r, cols), lambda i: (0, 0))


_tc_first = pl.pallas_call(
    _tc_first_body,
    grid=(_GRID,),
    in_specs=[_rows_spec(D), _full_spec(D, D), _rows_spec(16)],
    out_specs=_rows_spec(D),
    out_shape=jax.ShapeDtypeStruct((NPAD, D), jnp.float32),
)

_tc_mid = pl.pallas_call(
    _tc_mid_body,
    grid=(_GRID,),
    in_specs=[_rows_spec(16), _rows_spec(D), _rows_spec(D),
              _full_spec(D, D), _full_spec(1, D)],
    out_specs=_rows_spec(D),
    out_shape=jax.ShapeDtypeStruct((NPAD, D), jnp.float32),
)

_tc_last = pl.pallas_call(
    _tc_last_body,
    grid=(_GRID,),
    in_specs=[_rows_spec(16), _rows_spec(D), _rows_spec(D), _full_spec(1, D)],
    out_specs=_rows_spec(D),
    out_shape=jax.ShapeDtypeStruct((NPAD, D), jnp.float32),
)


def kernel(x, edge_index, W1, b1, W2, b2, W3, b3):
    src = edge_index[0].astype(jnp.int32)
    dst = edge_index[1].astype(jnp.int32)

    dst_s, src_s = lax.sort((dst, src), num_keys=1)
    starts = jnp.searchsorted(
        dst_s, (jnp.arange(NW) * RPB).astype(dst_s.dtype)).astype(jnp.int32)
    bnd = jnp.concatenate(
        [starts, jnp.full((1,), E, jnp.int32), jnp.zeros((16,), jnp.int32)])
    npe = EPAD - E
    src_p = jnp.concatenate([src_s, jnp.zeros((npe,), jnp.int32)])
    dst_p = jnp.concatenate([dst_s, jnp.full((npe,), N, jnp.int32)])

    xp = jnp.pad(x, ((0, NPAD - N), (0, 0)))
    b1r, b2r, b3r = (b.reshape(1, D) for b in (b1, b2, b3))

    degp = _deg_kernel(dst_p, bnd)
    g1 = _tc_first(xp, W1, degp)
    s1 = _spmm_kernel(g1, src_p, dst_p, bnd)
    g2 = _tc_mid(degp, s1, g1, W2, b1r)
    s2 = _spmm_kernel(g2, src_p, dst_p, bnd)
    g3 = _tc_mid(degp, s2, g2, W3, b2r)
    s3 = _spmm_kernel(g3, src_p, dst_p, bnd)
    out = _tc_last(degp, s3, g3, b3r)
    return out[:N]
